# baseline (speedup 1.0000x reference)
import functools

import jax
import jax.numpy as jnp
from jax import lax
from jax.experimental import pallas as pl
from jax.experimental.pallas import tpu as pltpu
from jax.experimental.pallas import tpu_sc as plsc

N = 10000
E = 320000
F = 128
NG = 64

NC = 2
NS = 16
NW = NC * NS
L = 16

E_PER_W = E // NW


def _deg_kernel_body(dst_hbm, deg_hbm, idx_v, ones_v, zeros_v, deg_sh):
    cid = lax.axis_index("c")
    sid = lax.axis_index("s")
    wid = cid * NS + sid
    base = wid * E_PER_W

    def fill(i, _):
        ones_v[pl.ds(i * L, L)] = jnp.full((L,), 1.0, jnp.float32)
        return ()
    lax.fori_loop(0, E_PER_W // L, fill, ())

    @pl.when(sid == 0)
    def _():
        def zfill(i, _):
            zeros_v[pl.ds(i * L, L)] = jnp.zeros((L,), jnp.float32)
            return ()
        lax.fori_loop(0, N // L, zfill, ())
        pltpu.sync_copy(zeros_v, deg_sh)

    plsc.subcore_barrier()

    pltpu.sync_copy(dst_hbm.at[pl.ds(base, E_PER_W)], idx_v)
    pltpu.sync_copy(ones_v, deg_sh.at[idx_v], add=True)

    plsc.subcore_barrier()

    @pl.when(sid == 0)
    def _():
        pltpu.sync_copy(deg_sh, deg_hbm.at[cid])


def _deg_partials(dst):
    mesh = plsc.VectorSubcoreMesh(core_axis_name="c", subcore_axis_name="s")
    return pl.kernel(
        _deg_kernel_body,
        out_type=jax.ShapeDtypeStruct((NC, N), jnp.float32),
        mesh=mesh,
        scratch_types=[
            pltpu.VMEM((E_PER_W,), jnp.int32),
            pltpu.VMEM((E_PER_W,), jnp.float32),
            pltpu.VMEM((N,), jnp.float32),
            pltpu.VMEM_SHARED((N,), jnp.float32),
        ],
    )(dst)


K = 40
NCHUNKS = E_PER_W // K
ROW_STRIDE = 624
ROW_SPAN = 640


NSLOT = 4


def _agg_body(y_hbm, srcw_hbm, dstw_hbm, zeros_hbm, acc_hbm,
              src_v, dst_v, b0, b1, b2, b3,
              g0, g1, g2, g3, s0, s1, s2, s3, acc_sh):
    bufs = (b0, b1, b2, b3)
    gsems = (g0, g1, g2, g3)
    ssems = (s0, s1, s2, s3)
    cid = lax.axis_index("c")
    sid = lax.axis_index("s")
    wid = cid * NS + sid

    rbase = sid * ROW_STRIDE
    pltpu.sync_copy(zeros_hbm.at[pl.ds(rbase, ROW_SPAN)],
                    acc_sh.at[pl.ds(rbase, ROW_SPAN)])

    ebase = wid * E_PER_W
    pltpu.sync_copy(srcw_hbm.at[pl.ds(ebase, E_PER_W)], src_v)
    pltpu.sync_copy(dstw_hbm.at[pl.ds(ebase, E_PER_W)], dst_v)

    plsc.subcore_barrier()

    def gather(j, s):
        pltpu.async_copy(y_hbm.at[src_v.at[pl.ds(j * K, K)]], bufs[s], gsems[s])

    def gwait(j, s):
        pltpu.make_async_copy(y_hbm.at[src_v.at[pl.ds(j * K, K)]], bufs[s],
                              gsems[s]).wait()

    def scat(j, s):
        pltpu.async_copy(bufs[s], acc_sh.at[dst_v.at[pl.ds(j * K, K)]],
                         ssems[s], add=True)

    def swait(j, s):
        pltpu.make_async_copy(bufs[s], acc_sh.at[dst_v.at[pl.ds(j * K, K)]],
                              ssems[s]).wait()

    for s in range(NSLOT):
        gather(s, s)

    def quad(i, _):
        j = i * NSLOT
        for s in range(NSLOT):
            gwait(j + s, s)
            scat(j + s, s)
            swait(j + s, s)

            @pl.when(j + s + NSLOT < NCHUNKS)
            def _():
                gather(j + s + NSLOT, s)
        return ()
    lax.fori_loop(0, NCHUNKS // NSLOT, quad, ())

    for t in range(NCHUNKS - (NCHUNKS // NSLOT) * NSLOT):
        j = (NCHUNKS // NSLOT) * NSLOT + t
        gwait(j, t)
        scat(j, t)
        swait(j, t)

    plsc.subcore_barrier()

    pltpu.sync_copy(acc_sh.at[pl.ds(rbase, ROW_SPAN)],
                    acc_hbm.at[cid, pl.ds(rbase, ROW_SPAN)])


def _agg_partials(y, src, dst):
    zeros = jnp.zeros((N, F), jnp.float32)
    mesh = plsc.VectorSubcoreMesh(core_axis_name="c", subcore_axis_name="s")
    return pl.kernel(
        _agg_body,
        out_type=jax.ShapeDtypeStruct((NC, N, F), jnp.float32),
        mesh=mesh,
        scratch_types=(
            [pltpu.VMEM((E_PER_W,), jnp.int32),
             pltpu.VMEM((E_PER_W,), jnp.int32)]
            + [pltpu.VMEM((K, F), jnp.float32)] * NSLOT
            + [pltpu.SemaphoreType.DMA] * (2 * NSLOT)
            + [pltpu.VMEM_SHARED((N, F), jnp.float32)]
        ),
    )(y, src, dst, zeros)


RC = 80
NRC = N // RC
_GDN = lax.GatherDimensionNumbers(
    offset_dims=(), collapsed_slice_dims=(0,), start_index_map=(0,))


def _lane_bcast(vec, lane_vec):
    return lax.gather(vec, lane_vec[:, None], _GDN, slice_sizes=(1,),
                      mode=lax.GatherScatterMode.PROMISE_IN_BOUNDS)


NPC = (NRC + NW - 1) // NW


def _pool_body(acc0f, acc1f, yf, dinv_hbm, batch_hbm, bc_hbm,
               pmax_hbm, psum_hbm,
               a0_0, a0_1, a1_0, a1_1, y_0, y_1, dv_0, dv_1, bt_0, bt_1,
               sm_0, sm_1, bc_v, pm_v, ps_v):
    a0s, a1s, ys = (a0_0, a0_1), (a1_0, a1_1), (y_0, y_1)
    dvs, bts, sms = (dv_0, dv_1), (bt_0, bt_1), (sm_0, sm_1)
    cid = lax.axis_index("c")
    sid = lax.axis_index("s")
    wid = cid * NS + sid

    pltpu.sync_copy(bc_hbm, bc_v)

    def copies(i, s):
        base = (wid + NW * i) * RC
        return (
            (acc0f.at[pl.ds(base * F, RC * F)], a0s[s]),
            (acc1f.at[pl.ds(base * F, RC * F)], a1s[s]),
            (yf.at[pl.ds(base * F, RC * F)], ys[s]),
            (dinv_hbm.at[pl.ds(base, RC)], dvs[s].at[pl.ds(0, RC)]),
            (batch_hbm.at[pl.ds(base, RC)], bts[s].at[pl.ds(0, RC)]),
        )

    def load(i, s):
        @pl.when(wid + NW * i < NRC)
        def _():
            for src, dstb in copies(i, s):
                pltpu.async_copy(src, dstb, sms[s])

    def loadwait(i, s):
        @pl.when(wid + NW * i < NRC)
        def _():
            for src, dstb in copies(i, s):
                pltpu.make_async_copy(src, dstb, sms[s]).wait()

    def pinit(i, _):
        pm_v[pl.ds(i * L, L)] = jnp.full((L,), -jnp.inf, jnp.float32)
        ps_v[pl.ds(i * L, L)] = jnp.zeros((L,), jnp.float32)
        return ()

    load(0, 0)
    lax.fori_loop(0, (NG * F) // L, pinit, ())

    for i in range(NPC):
        s = i % 2
        if i + 1 < NPC:
            load(i + 1, (i + 1) % 2)
        loadwait(i, s)

        @pl.when(wid + NW * i < NRC)
        def _():
            a0_v, a1_v, y_v, dv_v, bt_v = a0s[s], a1s[s], ys[s], dvs[s], bts[s]

            def row(r, _):
                dvr = jnp.full((L,), dv_v[pl.ds(r, L)][0], jnp.float32)
                ioff = bt_v[pl.ds(r, L)][0] * F + lax.iota(jnp.int32, L)
                for g in range(F // L):
                    o = r * F + g * L
                    t = a0_v[pl.ds(o, L)] + a1_v[pl.ds(o, L)] + y_v[pl.ds(o, L)]
                    h = jnp.maximum(dvr * t + bc_v[pl.ds(g * L, L)], 0.0)
                    idx = ioff + g * L
                    cur = plsc.load_gather(pm_v, [idx])
                    plsc.store_scatter(pm_v, [idx], jnp.maximum(cur, h))
                    plsc.addupdate_scatter(ps_v, [idx], h)
                return ()
            lax.fori_loop(0, RC, row, ())

    pltpu.sync_copy(pm_v, pmax_hbm.at[wid])
    pltpu.sync_copy(ps_v, psum_hbm.at[wid])


def _pool_partials(acc, y, dinv, batch, bc):
    acc0f = acc[0].reshape(N * F)
    acc1f = acc[1].reshape(N * F)
    yf = y.reshape(N * F)
    mesh = plsc.VectorSubcoreMesh(core_axis_name="c", subcore_axis_name="s")
    return pl.kernel(
        _pool_body,
        out_type=(jax.ShapeDtypeStruct((NW, NG * F), jnp.float32),
                  jax.ShapeDtypeStruct((NW, NG * F), jnp.float32)),
        mesh=mesh,
        compiler_params=pltpu.CompilerParams(needs_layout_passes=False),
        scratch_types=(
            [pltpu.VMEM((RC * F,), jnp.float32)] * 6
            + [pltpu.VMEM((RC + L,), jnp.float32)] * 2
            + [pltpu.VMEM((RC + L,), jnp.int32)] * 2
            + [pltpu.SemaphoreType.DMA] * 2
            + [pltpu.VMEM((F,), jnp.float32),
               pltpu.VMEM((NG * F,), jnp.float32),
               pltpu.VMEM((NG * F,), jnp.float32)]
        ),
    )(acc0f, acc1f, yf, dinv, batch, bc)


_MMB = 1000


def _scale_body(x_ref, wc_ref, degp_ref, y_ref, dinv_ref):
    deg = 1.0 + degp_ref[0, :, 0] + degp_ref[1, :, 0]
    dinv = lax.rsqrt(deg)
    xw = jnp.dot(x_ref[...], wc_ref[...], preferred_element_type=jnp.float32)
    y_ref[...] = xw * dinv[:, None]
    dinv_ref[...] = dinv[:, None]


def _matmul_scale(x, Wc, degp):
    y, dinv2 = pl.pallas_call(
        _scale_body,
        grid=(N // _MMB,),
        in_specs=[
            pl.BlockSpec((_MMB, F), lambda i: (i, 0)),
            pl.BlockSpec((F, F), lambda i: (0, 0)),
            pl.BlockSpec((2, _MMB, 1), lambda i: (0, i, 0)),
        ],
        out_specs=(pl.BlockSpec((_MMB, F), lambda i: (i, 0)),
                   pl.BlockSpec((_MMB, 1), lambda i: (i, 0))),
        out_shape=(jax.ShapeDtypeStruct((N, F), jnp.float32),
                   jax.ShapeDtypeStruct((N, 1), jnp.float32)),
    )(x, Wc, degp.reshape(2, N, 1))
    return y, dinv2.reshape(N)


def _head_body(pmax_ref, psum_ref, batch_ref, w1_ref, b1_ref, w2_ref, b2_ref,
               w3_ref, b3_ref, out_ref):
    gmax = jnp.max(pmax_ref[...], axis=0)
    gsum = jnp.sum(psum_ref[...], axis=0)
    onehot = (batch_ref[...][:, None] ==
              lax.broadcasted_iota(jnp.int32, (1, NG), 1)).astype(jnp.float32)
    cnt = jnp.sum(onehot, axis=0)
    gmean = gsum / jnp.maximum(cnt, 1.0)[:, None]
    out = jax.nn.relu(jnp.concatenate([gmax, gmean], axis=1))
    out = jax.nn.relu(
        jnp.dot(out, w1_ref[...], preferred_element_type=jnp.float32)
        + b1_ref[...])
    out = jax.nn.relu(
        jnp.dot(out, w2_ref[...], preferred_element_type=jnp.float32)
        + b2_ref[...])
    out = (jnp.dot(out, w3_ref[...], preferred_element_type=jnp.float32)
           + b3_ref[...])
    shifted = out - jnp.max(out, axis=-1, keepdims=True)
    lse = jnp.log(jnp.sum(jnp.exp(shifted), axis=-1, keepdims=True))
    out_ref[...] = shifted - lse


def _head(pmaxp, psump, batch, W1, b1, W2, b2, W3, b3):
    pmax3 = pmaxp.reshape(NW, NG, F)
    psum3 = psump.reshape(NW, NG, F)
    ncls = W3.shape[1]
    return pl.pallas_call(
        _head_body,
        out_shape=jax.ShapeDtypeStruct((NG, ncls), jnp.float32),
    )(pmax3, psum3, batch, W1, b1, W2, b2, W3, b3)


def kernel(x, edge_index, batch, Wc, bc, W1, b1, W2, b2, W3, b3):
    src = edge_index[0]
    dst = edge_index[1]

    degp = _deg_partials(dst)
    y, dinv = _matmul_scale(x, Wc, degp)
    accp = _agg_partials(y, src, dst)
    pmaxp, psump = _pool_partials(accp, y, dinv, batch, bc)
    return _head(pmaxp, psump, batch, W1, b1, W2, b2, W3, b3)


# speedup vs baseline: 1.0693x; 1.0693x over previous
import functools

import jax
import jax.numpy as jnp
from jax import lax
from jax.experimental import pallas as pl
from jax.experimental.pallas import tpu as pltpu
from jax.experimental.pallas import tpu_sc as plsc

N = 10000
E = 320000
F = 128
NG = 64

NC = 2
NS = 16
NW = NC * NS
L = 16

E_PER_W = E // NW


def _deg_kernel_body(dst_hbm, deg_hbm, idx_v, ones_v, zeros_v, deg_sh):
    cid = lax.axis_index("c")
    sid = lax.axis_index("s")
    wid = cid * NS + sid
    base = wid * E_PER_W

    def fill(i, _):
        ones_v[pl.ds(i * L, L)] = jnp.full((L,), 1.0, jnp.float32)
        return ()
    lax.fori_loop(0, E_PER_W // L, fill, ())

    @pl.when(sid == 0)
    def _():
        def zfill(i, _):
            zeros_v[pl.ds(i * L, L)] = jnp.zeros((L,), jnp.float32)
            return ()
        lax.fori_loop(0, N // L, zfill, ())
        pltpu.sync_copy(zeros_v, deg_sh)

    plsc.subcore_barrier()

    pltpu.sync_copy(dst_hbm.at[pl.ds(base, E_PER_W)], idx_v)
    pltpu.sync_copy(ones_v, deg_sh.at[idx_v], add=True)

    plsc.subcore_barrier()

    @pl.when(sid == 0)
    def _():
        pltpu.sync_copy(deg_sh, deg_hbm.at[cid])


def _deg_partials(dst):
    mesh = plsc.VectorSubcoreMesh(core_axis_name="c", subcore_axis_name="s")
    return pl.kernel(
        _deg_kernel_body,
        out_type=jax.ShapeDtypeStruct((NC, N), jnp.float32),
        mesh=mesh,
        scratch_types=[
            pltpu.VMEM((E_PER_W,), jnp.int32),
            pltpu.VMEM((E_PER_W,), jnp.float32),
            pltpu.VMEM((N,), jnp.float32),
            pltpu.VMEM_SHARED((N,), jnp.float32),
        ],
    )(dst)


K = 40
NCHUNKS = E_PER_W // K
ROW_STRIDE = 624
ROW_SPAN = 640


NSLOT = 4


def _agg_body(y_hbm, srcw_hbm, dstw_hbm, zeros_hbm, acc_hbm,
              src_v, dst_v, b0, b1, b2, b3,
              g0, g1, g2, g3, s0, s1, s2, s3, acc_sh):
    bufs = (b0, b1, b2, b3)
    gsems = (g0, g1, g2, g3)
    ssems = (s0, s1, s2, s3)
    cid = lax.axis_index("c")
    sid = lax.axis_index("s")
    wid = cid * NS + sid

    rbase = sid * ROW_STRIDE
    pltpu.sync_copy(zeros_hbm.at[pl.ds(rbase, ROW_SPAN)],
                    acc_sh.at[pl.ds(rbase, ROW_SPAN)])

    ebase = wid * E_PER_W
    pltpu.sync_copy(srcw_hbm.at[pl.ds(ebase, E_PER_W)], src_v)
    pltpu.sync_copy(dstw_hbm.at[pl.ds(ebase, E_PER_W)], dst_v)

    plsc.subcore_barrier()

    def gather(j, s):
        pltpu.async_copy(y_hbm.at[src_v.at[pl.ds(j * K, K)]], bufs[s], gsems[s])

    def gwait(j, s):
        pltpu.make_async_copy(y_hbm.at[src_v.at[pl.ds(j * K, K)]], bufs[s],
                              gsems[s]).wait()

    def scat(j, s):
        pltpu.async_copy(bufs[s], acc_sh.at[dst_v.at[pl.ds(j * K, K)]],
                         ssems[s], add=True)

    def swait(j, s):
        pltpu.make_async_copy(bufs[s], acc_sh.at[dst_v.at[pl.ds(j * K, K)]],
                              ssems[s]).wait()

    for s in range(NSLOT):
        gather(s, s)

    def quad(i, _):
        j = i * NSLOT
        for s in range(NSLOT):
            gwait(j + s, s)
            scat(j + s, s)
            swait(j + s, s)

            @pl.when(j + s + NSLOT < NCHUNKS)
            def _():
                gather(j + s + NSLOT, s)
        return ()
    lax.fori_loop(0, NCHUNKS // NSLOT, quad, ())

    for t in range(NCHUNKS - (NCHUNKS // NSLOT) * NSLOT):
        j = (NCHUNKS // NSLOT) * NSLOT + t
        gwait(j, t)
        scat(j, t)
        swait(j, t)

    plsc.subcore_barrier()

    pltpu.sync_copy(acc_sh.at[pl.ds(rbase, ROW_SPAN)],
                    acc_hbm.at[cid, pl.ds(rbase, ROW_SPAN)])


def _agg_partials(y, src, dst):
    zeros = jnp.zeros((N, F), jnp.float32)
    mesh = plsc.VectorSubcoreMesh(core_axis_name="c", subcore_axis_name="s")
    return pl.kernel(
        _agg_body,
        out_type=jax.ShapeDtypeStruct((NC, N, F), jnp.float32),
        mesh=mesh,
        scratch_types=(
            [pltpu.VMEM((E_PER_W,), jnp.int32),
             pltpu.VMEM((E_PER_W,), jnp.int32)]
            + [pltpu.VMEM((K, F), jnp.float32)] * NSLOT
            + [pltpu.SemaphoreType.DMA] * (2 * NSLOT)
            + [pltpu.VMEM_SHARED((N, F), jnp.float32)]
        ),
    )(y, src, dst, zeros)


RC = 80
NRC = N // RC
_GDN = lax.GatherDimensionNumbers(
    offset_dims=(), collapsed_slice_dims=(0,), start_index_map=(0,))


def _lane_bcast(vec, lane_vec):
    return lax.gather(vec, lane_vec[:, None], _GDN, slice_sizes=(1,),
                      mode=lax.GatherScatterMode.PROMISE_IN_BOUNDS)


NPC = (NRC + NW - 1) // NW


def _pool_body(acc0f, acc1f, yf, dinv_hbm, batch_hbm, bc_hbm,
               pmax_hbm, psum_hbm,
               a0_0, a0_1, a1_0, a1_1, y_0, y_1, dv_0, dv_1, bt_0, bt_1,
               sm_0, sm_1, bc_v, pm_v, ps_v):
    a0s, a1s, ys = (a0_0, a0_1), (a1_0, a1_1), (y_0, y_1)
    dvs, bts, sms = (dv_0, dv_1), (bt_0, bt_1), (sm_0, sm_1)
    cid = lax.axis_index("c")
    sid = lax.axis_index("s")
    wid = cid * NS + sid

    pltpu.sync_copy(bc_hbm, bc_v)

    def copies(i, s):
        base = (wid + NW * i) * RC
        return (
            (acc0f.at[pl.ds(base * F, RC * F)], a0s[s]),
            (acc1f.at[pl.ds(base * F, RC * F)], a1s[s]),
            (yf.at[pl.ds(base * F, RC * F)], ys[s]),
            (dinv_hbm.at[pl.ds(base, RC)], dvs[s].at[pl.ds(0, RC)]),
            (batch_hbm.at[pl.ds(base, RC)], bts[s].at[pl.ds(0, RC)]),
        )

    def load(i, s):
        @pl.when(wid + NW * i < NRC)
        def _():
            for src, dstb in copies(i, s):
                pltpu.async_copy(src, dstb, sms[s])

    def loadwait(i, s):
        @pl.when(wid + NW * i < NRC)
        def _():
            for src, dstb in copies(i, s):
                pltpu.make_async_copy(src, dstb, sms[s]).wait()

    def pinit(i, _):
        pm_v[pl.ds(i * L, L)] = jnp.full((L,), -jnp.inf, jnp.float32)
        ps_v[pl.ds(i * L, L)] = jnp.zeros((L,), jnp.float32)
        return ()

    load(0, 0)
    lax.fori_loop(0, (NG * F) // L, pinit, ())

    for i in range(NPC):
        s = i % 2
        if i + 1 < NPC:
            load(i + 1, (i + 1) % 2)
        loadwait(i, s)

        @pl.when(wid + NW * i < NRC)
        def _():
            a0_v, a1_v, y_v, dv_v, bt_v = a0s[s], a1s[s], ys[s], dvs[s], bts[s]
            bcs = [bc_v[pl.ds(g * L, L)] for g in range(F // L)]
            iot = lax.iota(jnp.int32, L)

            def row(r, _):
                dvr = jnp.full((L,), dv_v[pl.ds(r, L)][0], jnp.float32)
                ioff = bt_v[pl.ds(r, L)][0] * F + iot
                for g in range(F // L):
                    o = r * F + g * L
                    t = a0_v[pl.ds(o, L)] + a1_v[pl.ds(o, L)] + y_v[pl.ds(o, L)]
                    h = jnp.maximum(dvr * t + bcs[g], 0.0)
                    idx = ioff + g * L
                    cur = plsc.load_gather(pm_v, [idx])
                    plsc.store_scatter(pm_v, [idx], jnp.maximum(cur, h))
                    plsc.addupdate_scatter(ps_v, [idx], h)
                return ()
            lax.fori_loop(0, RC, row, ())

    pltpu.sync_copy(pm_v, pmax_hbm.at[wid])
    pltpu.sync_copy(ps_v, psum_hbm.at[wid])


def _pool_partials(acc, y, dinv, batch, bc):
    acc0f = acc[0].reshape(N * F)
    acc1f = acc[1].reshape(N * F)
    yf = y.reshape(N * F)
    mesh = plsc.VectorSubcoreMesh(core_axis_name="c", subcore_axis_name="s")
    return pl.kernel(
        _pool_body,
        out_type=(jax.ShapeDtypeStruct((NW, NG * F), jnp.float32),
                  jax.ShapeDtypeStruct((NW, NG * F), jnp.float32)),
        mesh=mesh,
        compiler_params=pltpu.CompilerParams(needs_layout_passes=False),
        scratch_types=(
            [pltpu.VMEM((RC * F,), jnp.float32)] * 6
            + [pltpu.VMEM((RC + L,), jnp.float32)] * 2
            + [pltpu.VMEM((RC + L,), jnp.int32)] * 2
            + [pltpu.SemaphoreType.DMA] * 2
            + [pltpu.VMEM((F,), jnp.float32),
               pltpu.VMEM((NG * F,), jnp.float32),
               pltpu.VMEM((NG * F,), jnp.float32)]
        ),
    )(acc0f, acc1f, yf, dinv, batch, bc)


def _scale_body(x_ref, wc_ref, degp_ref, y_ref, dinv_ref):
    deg = 1.0 + degp_ref[0, :] + degp_ref[1, :]
    dinv = lax.rsqrt(deg)
    xw = jnp.dot(x_ref[...], wc_ref[...], preferred_element_type=jnp.float32)
    y_ref[...] = xw * dinv[:, None]
    dinv_ref[...] = dinv


def _matmul_scale(x, Wc, degp):
    return pl.pallas_call(
        _scale_body,
        out_shape=(jax.ShapeDtypeStruct((N, F), jnp.float32),
                   jax.ShapeDtypeStruct((N,), jnp.float32)),
    )(x, Wc, degp)


def _head_body(pmax_ref, psum_ref, batch_ref, w1_ref, b1_ref, w2_ref, b2_ref,
               w3_ref, b3_ref, out_ref):
    gmax = jnp.max(pmax_ref[...], axis=0)
    gsum = jnp.sum(psum_ref[...], axis=0)
    onehot = (batch_ref[...][:, None] ==
              lax.broadcasted_iota(jnp.int32, (1, NG), 1)).astype(jnp.float32)
    cnt = jnp.sum(onehot, axis=0)
    gmean = gsum / jnp.maximum(cnt, 1.0)[:, None]
    out = jax.nn.relu(jnp.concatenate([gmax, gmean], axis=1))
    out = jax.nn.relu(
        jnp.dot(out, w1_ref[...], preferred_element_type=jnp.float32)
        + b1_ref[...])
    out = jax.nn.relu(
        jnp.dot(out, w2_ref[...], preferred_element_type=jnp.float32)
        + b2_ref[...])
    out = (jnp.dot(out, w3_ref[...], preferred_element_type=jnp.float32)
           + b3_ref[...])
    shifted = out - jnp.max(out, axis=-1, keepdims=True)
    lse = jnp.log(jnp.sum(jnp.exp(shifted), axis=-1, keepdims=True))
    out_ref[...] = shifted - lse


def _head(pmaxp, psump, batch, W1, b1, W2, b2, W3, b3):
    pmax3 = pmaxp.reshape(NW, NG, F)
    psum3 = psump.reshape(NW, NG, F)
    ncls = W3.shape[1]
    return pl.pallas_call(
        _head_body,
        out_shape=jax.ShapeDtypeStruct((NG, ncls), jnp.float32),
    )(pmax3, psum3, batch, W1, b1, W2, b2, W3, b3)


def kernel(x, edge_index, batch, Wc, bc, W1, b1, W2, b2, W3, b3):
    src = edge_index[0]
    dst = edge_index[1]

    degp = _deg_partials(dst)
    y, dinv = _matmul_scale(x, Wc, degp)
    accp = _agg_partials(y, src, dst)
    pmaxp, psump = _pool_partials(accp, y, dinv, batch, bc)
    return _head(pmaxp, psump, batch, W1, b1, W2, b2, W3, b3)
